# Initial kernel scaffold; baseline (speedup 1.0000x reference)
import jax
import jax.numpy as jnp
from jax.experimental import pallas as pl


def kernel(features, edge_index, W1, b1, F_mat, embeddings):
    raise NotImplementedError("write your pallas kernel here")


# speedup vs baseline: 12.3263x; 12.3263x over previous
import functools

import jax
import jax.numpy as jnp
from jax import lax
from jax.experimental import pallas as pl
from jax.experimental.pallas import tpu as pltpu
from jax.experimental.pallas import tpu_sc as plsc

N = 10000
E = 320000
D = 128

NC = 2
NS = 16
NPAD = 10240
RPT = NPAD // NS
K = 80

DEG_EPT = E // (NC * NS)
DEG_ITERS = DEG_EPT // K


def _deg_body(dst_hbm, ones_hbm, z_hbm, degp_hbm, didx, ones_v, acc, *, nc, ns):
    cid = lax.axis_index("c")
    sid = lax.axis_index("s")
    pltpu.sync_copy(ones_hbm, ones_v)
    pltpu.sync_copy(z_hbm, acc.at[pl.ds(sid * RPT, RPT)])
    plsc.subcore_barrier()

    def step(i, _):
        e0 = (cid * ns + sid) * DEG_EPT + i * K
        pltpu.sync_copy(dst_hbm.at[pl.ds(e0, K)], didx)
        pltpu.sync_copy(ones_v, acc.at[didx], add=True)
        return 0

    lax.fori_loop(0, DEG_ITERS, step, 0)
    plsc.subcore_barrier()
    pltpu.sync_copy(acc.at[pl.ds(sid * RPT, RPT)],
                    degp_hbm.at[cid, pl.ds(sid * RPT, RPT)])


def _sc_degree(dst):
    mesh = plsc.VectorSubcoreMesh(core_axis_name="c", subcore_axis_name="s")
    ones = jnp.ones((K, D), jnp.float32)
    zeros = jnp.zeros((RPT, D), jnp.float32)
    kern = pl.kernel(
        functools.partial(_deg_body, nc=NC, ns=NS),
        out_type=jax.ShapeDtypeStruct((NC, NPAD, D), jnp.float32),
        mesh=mesh,
        scratch_types=[
            pltpu.VMEM((K,), jnp.int32),
            pltpu.VMEM((K, D), jnp.float32),
            pltpu.VMEM_SHARED((NPAD, D), jnp.float32),
        ],
    )
    return kern(dst, ones, zeros)


SP_EPT = E // NS
SP_ITERS = SP_EPT // K


def _spmm_one(tab_hbm, src_hbm, dst_hbm, z_hbm, out_hbm,
              sidx, didx, rows, acc, sem, sid):
    pltpu.sync_copy(z_hbm, acc.at[pl.ds(sid * RPT, RPT)])
    plsc.subcore_barrier()

    def step(i, _):
        e0 = sid * SP_EPT + i * K
        pltpu.sync_copy(src_hbm.at[pl.ds(e0, K)], sidx)
        pltpu.sync_copy(dst_hbm.at[pl.ds(e0, K)], didx)
        pltpu.async_copy(tab_hbm.at[sidx], rows, sem).wait()
        pltpu.sync_copy(rows, acc.at[didx], add=True)
        return 0

    lax.fori_loop(0, SP_ITERS, step, 0)
    plsc.subcore_barrier()
    pltpu.sync_copy(acc.at[pl.ds(sid * RPT, RPT)],
                    out_hbm.at[pl.ds(sid * RPT, RPT)])


def _spmm_body(a_hbm, b_hbm, src_hbm, dst_hbm, z_hbm, s1_hbm, s2_hbm,
               sidx, didx, rows, acc, sem):
    cid = lax.axis_index("c")
    sid = lax.axis_index("s")

    @pl.when(cid == 0)
    def _():
        _spmm_one(a_hbm, src_hbm, dst_hbm, z_hbm, s1_hbm,
                  sidx, didx, rows, acc, sem, sid)

    @pl.when(cid == 1)
    def _():
        _spmm_one(b_hbm, src_hbm, dst_hbm, z_hbm, s2_hbm,
                  sidx, didx, rows, acc, sem, sid)


def _sc_spmm2(a, b, src, dst):
    mesh = plsc.VectorSubcoreMesh(core_axis_name="c", subcore_axis_name="s")
    zeros = jnp.zeros((RPT, D), jnp.float32)
    kern = pl.kernel(
        _spmm_body,
        out_type=[jax.ShapeDtypeStruct((NPAD, D), jnp.float32),
                  jax.ShapeDtypeStruct((NPAD, D), jnp.float32)],
        mesh=mesh,
        scratch_types=[
            pltpu.VMEM((K,), jnp.int32),
            pltpu.VMEM((K,), jnp.int32),
            pltpu.VMEM((K, D), jnp.float32),
            pltpu.VMEM_SHARED((NPAD, D), jnp.float32),
            pltpu.SemaphoreType.DMA,
        ],
    )
    return kern(a, b, src, dst, zeros)


RB = 1000


def _dinv_from(dega_blk, degb_blk):
    return lax.rsqrt(dega_blk + degb_blk + 1.0)


def _prep_body(feat, w1, emb, dega, degb, a_out, b_out):
    dinv = _dinv_from(dega[...], degb[...])
    xw = lax.dot_general(feat[...], w1[...], (((1,), (1,)), ((), ())),
                         preferred_element_type=jnp.float32)
    a_out[...] = dinv * xw
    b_out[...] = dinv * emb[...]


def _tc_prep(features, W1, embeddings, dega, degb):
    grid = (N // RB,)
    return pl.pallas_call(
        _prep_body,
        grid=grid,
        in_specs=[
            pl.BlockSpec((RB, D), lambda i: (i, 0)),
            pl.BlockSpec((D, D), lambda i: (0, 0)),
            pl.BlockSpec((RB, D), lambda i: (i, 0)),
            pl.BlockSpec((RB, D), lambda i: (i, 0)),
            pl.BlockSpec((RB, D), lambda i: (i, 0)),
        ],
        out_specs=[pl.BlockSpec((RB, D), lambda i: (i, 0)),
                   pl.BlockSpec((RB, D), lambda i: (i, 0))],
        out_shape=[jax.ShapeDtypeStruct((N, D), jnp.float32),
                   jax.ShapeDtypeStruct((N, D), jnp.float32)],
    )(features, W1, embeddings, dega, degb)


def _finish_body(s1, s2, a, b, dega, degb, f, b1, out):
    ftf = lax.dot_general(f[...], f[...], (((0,), (0,)), ((), ())),
                          preferred_element_type=jnp.float32)
    p = ftf / (jnp.sqrt(jnp.sum(ftf * ftf)) + 1e-5)
    dinv = _dinv_from(dega[...], degb[...])
    y = dinv * (s2[...] + b[...])
    h = jnp.maximum(dinv * (s1[...] + a[...]) + b1[...], 0.0)
    out[...] = 0.95 * lax.dot_general(y, p, (((1,), (0,)), ((), ())),
                                      preferred_element_type=jnp.float32) + h


def _tc_finish(s1, s2, a, b, dega, degb, F_mat, b1row):
    grid = (N // RB,)
    blk = pl.BlockSpec((RB, D), lambda i: (i, 0))
    return pl.pallas_call(
        _finish_body,
        grid=grid,
        in_specs=[blk, blk, blk, blk, blk, blk,
                  pl.BlockSpec((D, D), lambda i: (0, 0)),
                  pl.BlockSpec((1, D), lambda i: (0, 0))],
        out_specs=blk,
        out_shape=jax.ShapeDtypeStruct((N, D), jnp.float32),
    )(s1, s2, a, b, dega, degb, F_mat, b1row)


def kernel(features, edge_index, W1, b1, F_mat, embeddings):
    src = edge_index[0]
    dst = edge_index[1]
    degp = _sc_degree(dst)
    dega = degp[0, :N]
    degb = degp[1, :N]
    a, b = _tc_prep(features, W1, embeddings, dega, degb)
    s1, s2 = _sc_spmm2(a, b, src, dst)
    return _tc_finish(s1[:N], s2[:N], a, b, dega, degb, F_mat,
                      jnp.reshape(b1, (1, D)))
